# baseline (speedup 1.0000x reference)
import functools

import jax
import jax.numpy as jnp
from jax import lax
from jax.experimental import pallas as pl
from jax.experimental.pallas import tpu as pltpu
from jax.experimental.pallas import tpu_sc as plsc

_N = 10000
_E = 320000
_NHID = 128
_NHEADS = 8
_DHEAD = _NHID // _NHEADS
_ALPHA = 0.2

_NC = 2
_NS = 16
_NW = _NC * _NS
_EPW = _E // _NW
_CH = 40
_NCH = _EPW // _CH
_RPT = 640
_RPT_LAST = _N - (_NS - 1) * _RPT


def _prep_body(x_ref, wcat_ref, a1_ref, a2_ref, wh_ref, s1_ref, s2_ref):
    xb = x_ref[...]
    wh = jnp.dot(xb, wcat_ref[...], preferred_element_type=jnp.float32)
    wh_ref[...] = wh
    s1_ref[...] = jnp.dot(wh, a1_ref[...], preferred_element_type=jnp.float32)
    s2_ref[...] = jnp.dot(wh, a2_ref[...], preferred_element_type=jnp.float32)


def _prep(x, wcat, a1p, a2p):
    blk = 1000
    grid = (_N // blk,)
    return pl.pallas_call(
        _prep_body,
        grid=grid,
        in_specs=[
            pl.BlockSpec((blk, _NHID), lambda i: (i, 0)),
            pl.BlockSpec((_NHID, _NHID), lambda i: (0, 0)),
            pl.BlockSpec((_NHID, 16), lambda i: (0, 0)),
            pl.BlockSpec((_NHID, 16), lambda i: (0, 0)),
        ],
        out_specs=[
            pl.BlockSpec((blk, _NHID), lambda i: (i, 0)),
            pl.BlockSpec((blk, 16), lambda i: (i, 0)),
            pl.BlockSpec((blk, 16), lambda i: (i, 0)),
        ],
        out_shape=[
            jax.ShapeDtypeStruct((_N, _NHID), jnp.float32),
            jax.ShapeDtypeStruct((_N, 16), jnp.float32),
            jax.ShapeDtypeStruct((_N, 16), jnp.float32),
        ],
    )(x, wcat, a1p, a2p)


def _edge_body(wh_hbm, s1_hbm, s2_hbm, src_hbm, dst_hbm,
               msg_out, den_out,
               sidx0, didx0, s1v0, s2v0, whv0, ov0, gv0, didxs0,
               sidx1, didx1, s1v1, s2v1, whv1, ov1, gv1, didxs1,
               msg_acc, den_acc,
               semi0, semg0, sems0, semi1, semg1, sems1):
    c = lax.axis_index("c")
    s = lax.axis_index("s")

    z16 = jnp.zeros((16,), jnp.float32)

    @plsc.parallel_loop(0, _CH, step=1, unroll=4)
    def _zrow(e):
        for k in range(_NHID // 16):
            ov0[e, pl.ds(16 * k, 16)] = z16
        gv0[e, :] = z16

    r0 = s * _RPT
    nrep = _RPT // _CH

    @pl.when(s < _NS - 1)
    def _zero_full():
        for k in range(nrep):
            pltpu.sync_copy(ov0, msg_acc.at[pl.ds(r0 + k * _CH, _CH)])
            pltpu.sync_copy(gv0, den_acc.at[pl.ds(r0 + k * _CH, _CH)])

    @pl.when(s == _NS - 1)
    def _zero_last():
        for k in range(_RPT_LAST // _CH):
            pltpu.sync_copy(ov0, msg_acc.at[pl.ds(r0 + k * _CH, _CH)])
            pltpu.sync_copy(gv0, den_acc.at[pl.ds(r0 + k * _CH, _CH)])

    plsc.subcore_barrier()

    lanes = lax.iota(jnp.int32, 16)
    headmask = lanes < _NHEADS
    ebase = (c * _NS + s) * _EPW
    bufs = ((sidx0, didx0, s1v0, s2v0, whv0, semi0, semg0, ov0, gv0, didxs0, sems0),
            (sidx1, didx1, s1v1, s2v1, whv1, semi1, semg1, ov1, gv1, didxs1, sems1))

    def idx_start(j, b):
        off = jnp.minimum(ebase + j * _CH, _E - _CH)
        pltpu.async_copy(src_hbm.at[pl.ds(off, _CH)], b[0], b[5])
        pltpu.async_copy(dst_hbm.at[pl.ds(off, _CH)], b[1], b[5])

    def idx_wait(b):
        pltpu.make_async_copy(src_hbm.at[pl.ds(0, _CH)], b[0], b[5]).wait()
        pltpu.make_async_copy(dst_hbm.at[pl.ds(0, _CH)], b[1], b[5]).wait()

    def gat_start(b):
        pltpu.async_copy(s1_hbm.at[b[0]], b[2], b[6])
        pltpu.async_copy(s2_hbm.at[b[1]], b[3], b[6])
        pltpu.async_copy(wh_hbm.at[b[0]], b[4], b[6])

    def gat_wait(b):
        pltpu.make_async_copy(s1_hbm.at[b[0]], b[2], b[6]).wait()
        pltpu.make_async_copy(s2_hbm.at[b[1]], b[3], b[6]).wait()
        pltpu.make_async_copy(wh_hbm.at[b[0]], b[4], b[6]).wait()

    def compute(b):
        s1v, s2v, whv, ov, gv, didxs = b[2], b[3], b[4], b[7], b[8], b[9]

        _offs = list(range(0, _CH - 15, 16))
        if _CH % 16:
            _offs.append(_CH - 16)
        for k in _offs:
            didxs[pl.ds(k, 16)] = b[1][pl.ds(k, 16)]

        @plsc.parallel_loop(0, _CH, step=1, unroll=4)
        def _edges(e):
            t = s1v[e, :] + s2v[e, :]
            t = jnp.maximum(t, _ALPHA * t)
            g = jnp.where(headmask, jnp.exp(t), 0.0)
            gv[e, :] = g
            for h in range(_NHEADS):
                gh = lax.gather(
                    g, jnp.full((16, 1), h, jnp.int32),
                    lax.GatherDimensionNumbers(
                        offset_dims=(), collapsed_slice_dims=(0,),
                        start_index_map=(0,)),
                    slice_sizes=(1,),
                    mode=lax.GatherScatterMode.PROMISE_IN_BOUNDS)
                w = whv[e, pl.ds(_DHEAD * h, _DHEAD)]
                ov[e, pl.ds(_DHEAD * h, _DHEAD)] = w * gh

    def scat_start(b):
        pltpu.async_copy(b[7], msg_acc.at[b[9]], b[10], add=True)
        pltpu.async_copy(b[8], den_acc.at[b[9]], b[10], add=True)

    def scat_wait(b):
        pltpu.make_async_copy(b[7], msg_acc.at[b[9]], b[10]).wait()
        pltpu.make_async_copy(b[8], den_acc.at[b[9]], b[10]).wait()

    idx_start(0, bufs[0])
    idx_wait(bufs[0])
    gat_start(bufs[0])
    idx_start(1, bufs[1])

    def pair_body(jj, carry):
        j = jj * 2
        not_first = jj > 0
        idx_wait(bufs[1])
        gat_start(bufs[1])
        gat_wait(bufs[0])

        @pl.when(not_first)
        def _():
            scat_wait(bufs[0])

        compute(bufs[0])
        scat_start(bufs[0])
        idx_start(j + 2, bufs[0])
        idx_wait(bufs[0])
        gat_start(bufs[0])
        gat_wait(bufs[1])

        @pl.when(not_first)
        def _():
            scat_wait(bufs[1])

        compute(bufs[1])
        scat_start(bufs[1])
        idx_start(j + 3, bufs[1])
        return carry

    lax.fori_loop(0, (_NCH - 1) // 2, pair_body, 0)
    if _NCH % 2 == 1:
        idx_wait(bufs[1])
        gat_wait(bufs[0])
        scat_wait(bufs[0])
        compute(bufs[0])
        scat_start(bufs[0])
        scat_wait(bufs[1])
        scat_wait(bufs[0])
    else:
        idx_wait(bufs[1])
        gat_start(bufs[1])
        gat_wait(bufs[0])
        scat_wait(bufs[0])
        compute(bufs[0])
        scat_start(bufs[0])
        gat_wait(bufs[1])
        scat_wait(bufs[1])
        compute(bufs[1])
        scat_start(bufs[1])
        scat_wait(bufs[0])
        scat_wait(bufs[1])
    plsc.subcore_barrier()

    @pl.when(s < _NS - 1)
    def _out_full():
        pltpu.sync_copy(msg_acc.at[pl.ds(r0, _RPT)], msg_out.at[c, pl.ds(r0, _RPT)])
        pltpu.sync_copy(den_acc.at[pl.ds(r0, _RPT)], den_out.at[c, pl.ds(r0, _RPT)])

    @pl.when(s == _NS - 1)
    def _out_last():
        pltpu.sync_copy(msg_acc.at[pl.ds(r0, _RPT_LAST)],
                        msg_out.at[c, pl.ds(r0, _RPT_LAST)])
        pltpu.sync_copy(den_acc.at[pl.ds(r0, _RPT_LAST)],
                        den_out.at[c, pl.ds(r0, _RPT_LAST)])


def _edge(wh, s1p, s2p, src, dst):
    mesh = plsc.VectorSubcoreMesh(core_axis_name="c", subcore_axis_name="s")
    run = functools.partial(
        pl.kernel,
        mesh=mesh,
        out_type=(
            jax.ShapeDtypeStruct((_NC, _N, _NHID), jnp.float32),
            jax.ShapeDtypeStruct((_NC, _N, 16), jnp.float32),
        ),
        scratch_types=[
            pltpu.VMEM((_CH,), jnp.int32),
            pltpu.VMEM((_CH,), jnp.int32),
            pltpu.VMEM((_CH, 16), jnp.float32),
            pltpu.VMEM((_CH, 16), jnp.float32),
            pltpu.VMEM((_CH, _NHID), jnp.float32),
            pltpu.VMEM((_CH, _NHID), jnp.float32),
            pltpu.VMEM((_CH, 16), jnp.float32),
            pltpu.VMEM((_CH,), jnp.int32),
            pltpu.VMEM((_CH,), jnp.int32),
            pltpu.VMEM((_CH,), jnp.int32),
            pltpu.VMEM((_CH, 16), jnp.float32),
            pltpu.VMEM((_CH, 16), jnp.float32),
            pltpu.VMEM((_CH, _NHID), jnp.float32),
            pltpu.VMEM((_CH, _NHID), jnp.float32),
            pltpu.VMEM((_CH, 16), jnp.float32),
            pltpu.VMEM((_CH,), jnp.int32),
            pltpu.VMEM_SHARED((_N, _NHID), jnp.float32),
            pltpu.VMEM_SHARED((_N, 16), jnp.float32),
            pltpu.SemaphoreType.DMA,
            pltpu.SemaphoreType.DMA,
            pltpu.SemaphoreType.DMA,
            pltpu.SemaphoreType.DMA,
            pltpu.SemaphoreType.DMA,
            pltpu.SemaphoreType.DMA,
        ],
        compiler_params=pltpu.CompilerParams(use_tc_tiling_on_sc=False),
    )(_edge_body)
    return run(wh, s1p, s2p, src, dst)


def _gru_body(x_ref, msg_ref, den_ref, wih_ref, whh_ref, bih_ref, bhh_ref,
              rmat_ref, out_ref):
    xb = x_ref[...]
    msg = msg_ref[0] + msg_ref[1]
    den = den_ref[0] + den_ref[1]
    den_rep = jnp.dot(den, rmat_ref[...], preferred_element_type=jnp.float32)
    hcat = jnp.where(den_rep > 0.0, msg / den_rep, 0.0)
    gi = jnp.dot(xb, wih_ref[...], preferred_element_type=jnp.float32) + bih_ref[...]
    gh = jnp.dot(hcat, whh_ref[...], preferred_element_type=jnp.float32) + bhh_ref[...]
    r = jax.nn.sigmoid(gi[:, 0:_NHID] + gh[:, 0:_NHID])
    z = jax.nn.sigmoid(gi[:, _NHID:2 * _NHID] + gh[:, _NHID:2 * _NHID])
    n = jnp.tanh(gi[:, 2 * _NHID:] + r * gh[:, 2 * _NHID:])
    out_ref[...] = (1.0 - z) * n + z * hcat


def _gru(x, msg2, den2, wih, whh, bih, bhh, rmat):
    blk = 1000
    grid = (_N // blk,)
    return pl.pallas_call(
        _gru_body,
        grid=grid,
        in_specs=[
            pl.BlockSpec((blk, _NHID), lambda i: (i, 0)),
            pl.BlockSpec((_NC, blk, _NHID), lambda i: (0, i, 0)),
            pl.BlockSpec((_NC, blk, 16), lambda i: (0, i, 0)),
            pl.BlockSpec((_NHID, 3 * _NHID), lambda i: (0, 0)),
            pl.BlockSpec((_NHID, 3 * _NHID), lambda i: (0, 0)),
            pl.BlockSpec((1, 3 * _NHID), lambda i: (0, 0)),
            pl.BlockSpec((1, 3 * _NHID), lambda i: (0, 0)),
            pl.BlockSpec((16, _NHID), lambda i: (0, 0)),
        ],
        out_specs=pl.BlockSpec((blk, _NHID), lambda i: (i, 0)),
        out_shape=jax.ShapeDtypeStruct((_N, _NHID), jnp.float32),
    )(x, msg2, den2, wih, whh, bih, bhh, rmat)


def kernel(x, edge_index, W, a, W_ih, W_hh, b_ih, b_hh):
    wcat = jnp.transpose(W, (1, 0, 2)).reshape(_NHID, _NHID)
    a1 = a[:, :_DHEAD, 0]
    a2 = a[:, _DHEAD:, 0]
    eye = jnp.eye(_NHEADS, dtype=jnp.float32)
    a1p = jnp.pad((a1[:, :, None] * eye[:, None, :]).reshape(_NHID, _NHEADS),
                  ((0, 0), (0, 16 - _NHEADS)))
    a2p = jnp.pad((a2[:, :, None] * eye[:, None, :]).reshape(_NHID, _NHEADS),
                  ((0, 0), (0, 16 - _NHEADS)))
    rmat = (jnp.arange(_NHID)[None, :] // _DHEAD
            == jnp.arange(16)[:, None]).astype(jnp.float32)

    src = edge_index[0]
    dst = edge_index[1]

    wh, s1p, s2p = _prep(x, wcat, a1p, a2p)
    msg2, den2 = _edge(wh, s1p, s2p, src, dst)
    bih = b_ih.reshape(1, 3 * _NHID)
    bhh = b_hh.reshape(1, 3 * _NHID)
    return _gru(x, msg2, den2, W_ih, W_hh, bih, bhh, rmat)


# speedup vs baseline: 1.1817x; 1.1817x over previous
import functools

import jax
import jax.numpy as jnp
from jax import lax
from jax.experimental import pallas as pl
from jax.experimental.pallas import tpu as pltpu
from jax.experimental.pallas import tpu_sc as plsc

_N = 10000
_E = 320000
_NHID = 128
_NHEADS = 8
_DHEAD = _NHID // _NHEADS
_ALPHA = 0.2

_NC = 2
_NS = 16
_NW = _NC * _NS
_EPW = _E // _NW
_CH = 80
_NCH = _EPW // _CH
_STEADY = (_NCH - 5) // 6
_RPT = 640
_RPT_LAST = _N - (_NS - 1) * _RPT


def _prep_body(x_ref, wcat_ref, a1_ref, a2_ref, wh_ref, s1_ref, s2_ref):
    xb = x_ref[...]
    wh = jnp.dot(xb, wcat_ref[...], preferred_element_type=jnp.float32)
    wh_ref[...] = wh
    s1_ref[...] = jnp.dot(wh, a1_ref[...], preferred_element_type=jnp.float32)
    s2_ref[...] = jnp.dot(wh, a2_ref[...], preferred_element_type=jnp.float32)


def _prep(x, wcat, a1p, a2p):
    blk = 1000
    grid = (_N // blk,)
    return pl.pallas_call(
        _prep_body,
        grid=grid,
        in_specs=[
            pl.BlockSpec((blk, _NHID), lambda i: (i, 0)),
            pl.BlockSpec((_NHID, _NHID), lambda i: (0, 0)),
            pl.BlockSpec((_NHID, 16), lambda i: (0, 0)),
            pl.BlockSpec((_NHID, 16), lambda i: (0, 0)),
        ],
        out_specs=[
            pl.BlockSpec((blk, _NHID), lambda i: (i, 0)),
            pl.BlockSpec((blk, 16), lambda i: (i, 0)),
            pl.BlockSpec((blk, 16), lambda i: (i, 0)),
        ],
        out_shape=[
            jax.ShapeDtypeStruct((_N, _NHID), jnp.float32),
            jax.ShapeDtypeStruct((_N, 16), jnp.float32),
            jax.ShapeDtypeStruct((_N, 16), jnp.float32),
        ],
    )(x, wcat, a1p, a2p)


def _edge_body(wh_hbm, s1_hbm, s2_hbm, src_hbm, dst_hbm,
               msg_out, den_out,
               whv0, whv1, whv2,
               s1v0, s1v1, s2v0, s2v1, gv0, gv1,
               sidx0, sidx1, didx0, didx1, didxs0, didxs1,
               msg_acc, den_acc,
               semi0, semi1, semg0, semg1, semg2, sems0, sems1):
    c = lax.axis_index("c")
    s = lax.axis_index("s")

    WH = (whv0, whv1, whv2)
    SEMG = (semg0, semg1, semg2)
    S1 = (s1v0, s1v1)
    S2 = (s2v0, s2v1)
    GV = (gv0, gv1)
    SI = (sidx0, sidx1)
    DI = (didx0, didx1)
    DS = (didxs0, didxs1)
    SEMI = (semi0, semi1)
    SEMS = (sems0, sems1)

    z16 = jnp.zeros((16,), jnp.float32)

    @plsc.parallel_loop(0, _CH, step=1, unroll=4)
    def _zrow(e):
        for k in range(_NHID // 16):
            whv0[e, pl.ds(16 * k, 16)] = z16
        gv0[e, :] = z16

    r0 = s * _RPT

    @pl.when(s < _NS - 1)
    def _zero_full():
        for k in range(_RPT // _CH):
            pltpu.sync_copy(whv0, msg_acc.at[pl.ds(r0 + k * _CH, _CH)])
            pltpu.sync_copy(gv0, den_acc.at[pl.ds(r0 + k * _CH, _CH)])

    @pl.when(s == _NS - 1)
    def _zero_last():
        for k in range(_RPT_LAST // _CH):
            pltpu.sync_copy(whv0, msg_acc.at[pl.ds(r0 + k * _CH, _CH)])
            pltpu.sync_copy(gv0, den_acc.at[pl.ds(r0 + k * _CH, _CH)])

    plsc.subcore_barrier()

    lanes = lax.iota(jnp.int32, 16)
    headmask = lanes < _NHEADS
    ebase = (c * _NS + s) * _EPW

    def idx_start(j, r2):
        off = jnp.minimum(ebase + j * _CH, _E - _CH)
        pltpu.async_copy(src_hbm.at[pl.ds(off, _CH)], SI[r2], SEMI[r2])
        pltpu.async_copy(dst_hbm.at[pl.ds(off, _CH)], DI[r2], SEMI[r2])

    def idx_wait(r2):
        pltpu.make_async_copy(src_hbm.at[pl.ds(0, _CH)], SI[r2], SEMI[r2]).wait()
        pltpu.make_async_copy(dst_hbm.at[pl.ds(0, _CH)], DI[r2], SEMI[r2]).wait()

    def gat_start(r2, r3):
        pltpu.async_copy(s1_hbm.at[SI[r2]], S1[r2], SEMG[r3])
        pltpu.async_copy(s2_hbm.at[DI[r2]], S2[r2], SEMG[r3])
        pltpu.async_copy(wh_hbm.at[SI[r2]], WH[r3], SEMG[r3])

    def gat_wait(r2, r3):
        pltpu.make_async_copy(s1_hbm.at[SI[r2]], S1[r2], SEMG[r3]).wait()
        pltpu.make_async_copy(s2_hbm.at[DI[r2]], S2[r2], SEMG[r3]).wait()
        pltpu.make_async_copy(wh_hbm.at[SI[r2]], WH[r3], SEMG[r3]).wait()

    def compute(r2, r3):
        s1v, s2v, whv, gv, didxs = S1[r2], S2[r2], WH[r3], GV[r2], DS[r2]

        for k in range(0, _CH, 16):
            didxs[pl.ds(k, 16)] = DI[r2][pl.ds(k, 16)]

        @plsc.parallel_loop(0, _CH, step=1, unroll=4)
        def _edges(e):
            t = s1v[e, :] + s2v[e, :]
            t = jnp.maximum(t, _ALPHA * t)
            g = jnp.where(headmask, jnp.exp(t), 0.0)
            gv[e, :] = g
            for h in range(_NHEADS):
                gh = lax.gather(
                    g, jnp.full((16, 1), h, jnp.int32),
                    lax.GatherDimensionNumbers(
                        offset_dims=(), collapsed_slice_dims=(0,),
                        start_index_map=(0,)),
                    slice_sizes=(1,),
                    mode=lax.GatherScatterMode.PROMISE_IN_BOUNDS)
                w = whv[e, pl.ds(_DHEAD * h, _DHEAD)]
                whv[e, pl.ds(_DHEAD * h, _DHEAD)] = w * gh

    def scat_start(r2, r3):
        pltpu.async_copy(WH[r3], msg_acc.at[DS[r2]], SEMS[r2], add=True)
        pltpu.async_copy(GV[r2], den_acc.at[DS[r2]], SEMS[r2], add=True)

    def scat_wait(r2, r3):
        pltpu.make_async_copy(WH[r3], msg_acc.at[DS[r2]], SEMS[r2]).wait()
        pltpu.make_async_copy(GV[r2], den_acc.at[DS[r2]], SEMS[r2]).wait()

    def slot(jdyn, jo, do_next_gat, do_next_idx, guard_scat):
        r3, r2 = jo % 3, jo % 2
        r3n, r2n = (jo + 1) % 3, (jo + 1) % 2
        gat_wait(r2, r3)
        if guard_scat:
            @pl.when(jdyn >= 2)
            def _():
                scat_wait(r2, r3n)
        else:
            scat_wait(r2, r3n)
        if do_next_gat:
            idx_wait(r2n)
            gat_start(r2n, r3n)
        compute(r2, r3)
        scat_start(r2, r3)
        if do_next_idx:
            idx_start(jdyn + 2, r2)

    idx_start(0, 0)
    idx_wait(0)
    gat_start(0, 0)
    idx_start(1, 1)

    def six_body(k, carry):
        j = k * 6
        for jo in range(6):
            slot(j + jo, jo, True, True, True)
        return carry

    lax.fori_loop(0, _STEADY, six_body, 0)
    for jo in range(5):
        slot(_STEADY * 6 + jo, jo, jo < 4, jo < 3, False)
    scat_wait(1, 0)
    scat_wait(0, 1)
    plsc.subcore_barrier()

    @pl.when(s < _NS - 1)
    def _out_full():
        pltpu.sync_copy(msg_acc.at[pl.ds(r0, _RPT)], msg_out.at[c, pl.ds(r0, _RPT)])
        pltpu.sync_copy(den_acc.at[pl.ds(r0, _RPT)], den_out.at[c, pl.ds(r0, _RPT)])

    @pl.when(s == _NS - 1)
    def _out_last():
        pltpu.sync_copy(msg_acc.at[pl.ds(r0, _RPT_LAST)],
                        msg_out.at[c, pl.ds(r0, _RPT_LAST)])
        pltpu.sync_copy(den_acc.at[pl.ds(r0, _RPT_LAST)],
                        den_out.at[c, pl.ds(r0, _RPT_LAST)])


def _edge(wh, s1p, s2p, src, dst):
    mesh = plsc.VectorSubcoreMesh(core_axis_name="c", subcore_axis_name="s")
    run = functools.partial(
        pl.kernel,
        mesh=mesh,
        out_type=(
            jax.ShapeDtypeStruct((_NC, _N, _NHID), jnp.float32),
            jax.ShapeDtypeStruct((_NC, _N, 16), jnp.float32),
        ),
        scratch_types=[
            pltpu.VMEM((_CH, _NHID), jnp.float32),
            pltpu.VMEM((_CH, _NHID), jnp.float32),
            pltpu.VMEM((_CH, _NHID), jnp.float32),
            pltpu.VMEM((_CH, 16), jnp.float32),
            pltpu.VMEM((_CH, 16), jnp.float32),
            pltpu.VMEM((_CH, 16), jnp.float32),
            pltpu.VMEM((_CH, 16), jnp.float32),
            pltpu.VMEM((_CH, 16), jnp.float32),
            pltpu.VMEM((_CH, 16), jnp.float32),
            pltpu.VMEM((_CH,), jnp.int32),
            pltpu.VMEM((_CH,), jnp.int32),
            pltpu.VMEM((_CH,), jnp.int32),
            pltpu.VMEM((_CH,), jnp.int32),
            pltpu.VMEM((_CH,), jnp.int32),
            pltpu.VMEM((_CH,), jnp.int32),
            pltpu.VMEM_SHARED((_N, _NHID), jnp.float32),
            pltpu.VMEM_SHARED((_N, 16), jnp.float32),
            pltpu.SemaphoreType.DMA,
            pltpu.SemaphoreType.DMA,
            pltpu.SemaphoreType.DMA,
            pltpu.SemaphoreType.DMA,
            pltpu.SemaphoreType.DMA,
            pltpu.SemaphoreType.DMA,
            pltpu.SemaphoreType.DMA,
        ],
        compiler_params=pltpu.CompilerParams(use_tc_tiling_on_sc=False),
    )(_edge_body)
    return run(wh, s1p, s2p, src, dst)


def _gru_body(x_ref, msg_ref, den_ref, wih_ref, whh_ref, bih_ref, bhh_ref,
              rmat_ref, out_ref):
    xb = x_ref[...]
    msg = msg_ref[0] + msg_ref[1]
    den = den_ref[0] + den_ref[1]
    den_rep = jnp.dot(den, rmat_ref[...], preferred_element_type=jnp.float32)
    hcat = jnp.where(den_rep > 0.0, msg / den_rep, 0.0)
    gi = jnp.dot(xb, wih_ref[...], preferred_element_type=jnp.float32) + bih_ref[...]
    gh = jnp.dot(hcat, whh_ref[...], preferred_element_type=jnp.float32) + bhh_ref[...]
    r = jax.nn.sigmoid(gi[:, 0:_NHID] + gh[:, 0:_NHID])
    z = jax.nn.sigmoid(gi[:, _NHID:2 * _NHID] + gh[:, _NHID:2 * _NHID])
    n = jnp.tanh(gi[:, 2 * _NHID:] + r * gh[:, 2 * _NHID:])
    out_ref[...] = (1.0 - z) * n + z * hcat


def _gru(x, msg2, den2, wih, whh, bih, bhh, rmat):
    blk = 1000
    grid = (_N // blk,)
    return pl.pallas_call(
        _gru_body,
        grid=grid,
        in_specs=[
            pl.BlockSpec((blk, _NHID), lambda i: (i, 0)),
            pl.BlockSpec((_NC, blk, _NHID), lambda i: (0, i, 0)),
            pl.BlockSpec((_NC, blk, 16), lambda i: (0, i, 0)),
            pl.BlockSpec((_NHID, 3 * _NHID), lambda i: (0, 0)),
            pl.BlockSpec((_NHID, 3 * _NHID), lambda i: (0, 0)),
            pl.BlockSpec((1, 3 * _NHID), lambda i: (0, 0)),
            pl.BlockSpec((1, 3 * _NHID), lambda i: (0, 0)),
            pl.BlockSpec((16, _NHID), lambda i: (0, 0)),
        ],
        out_specs=pl.BlockSpec((blk, _NHID), lambda i: (i, 0)),
        out_shape=jax.ShapeDtypeStruct((_N, _NHID), jnp.float32),
    )(x, msg2, den2, wih, whh, bih, bhh, rmat)


def kernel(x, edge_index, W, a, W_ih, W_hh, b_ih, b_hh):
    wcat = jnp.transpose(W, (1, 0, 2)).reshape(_NHID, _NHID)
    a1 = a[:, :_DHEAD, 0]
    a2 = a[:, _DHEAD:, 0]
    eye = jnp.eye(_NHEADS, dtype=jnp.float32)
    a1p = jnp.pad((a1[:, :, None] * eye[:, None, :]).reshape(_NHID, _NHEADS),
                  ((0, 0), (0, 16 - _NHEADS)))
    a2p = jnp.pad((a2[:, :, None] * eye[:, None, :]).reshape(_NHID, _NHEADS),
                  ((0, 0), (0, 16 - _NHEADS)))
    rmat = (jnp.arange(_NHID)[None, :] // _DHEAD
            == jnp.arange(16)[:, None]).astype(jnp.float32)

    src = edge_index[0]
    dst = edge_index[1]

    wh, s1p, s2p = _prep(x, wcat, a1p, a2p)
    msg2, den2 = _edge(wh, s1p, s2p, src, dst)
    bih = b_ih.reshape(1, 3 * _NHID)
    bhh = b_hh.reshape(1, 3 * _NHID)
    return _gru(x, msg2, den2, W_ih, W_hh, bih, bhh, rmat)


# speedup vs baseline: 1.1842x; 1.0021x over previous
import functools

import jax
import jax.numpy as jnp
from jax import lax
from jax.experimental import pallas as pl
from jax.experimental.pallas import tpu as pltpu
from jax.experimental.pallas import tpu_sc as plsc

_N = 10000
_E = 320000
_NHID = 128
_NHEADS = 8
_DHEAD = _NHID // _NHEADS
_ALPHA = 0.2

_NC = 2
_NS = 16
_NW = _NC * _NS
_EPW = _E // _NW
_CH = 80
_NCH = _EPW // _CH
_STEADY = (_NCH - 5) // 6
_RPT = 640
_RPT_LAST = _N - (_NS - 1) * _RPT


def _prep_body(x_ref, wcat_ref, a1_ref, a2_ref, wh_ref, s1_ref, s2_ref):
    xb = x_ref[...]
    wh = jnp.dot(xb, wcat_ref[...], preferred_element_type=jnp.float32)
    wh_ref[...] = wh
    s1_ref[...] = jnp.dot(wh, a1_ref[...], preferred_element_type=jnp.float32)
    s2_ref[...] = jnp.dot(wh, a2_ref[...], preferred_element_type=jnp.float32)


def _prep(x, wcat, a1p, a2p):
    blk = 1000
    grid = (_N // blk,)
    return pl.pallas_call(
        _prep_body,
        grid=grid,
        in_specs=[
            pl.BlockSpec((blk, _NHID), lambda i: (i, 0)),
            pl.BlockSpec((_NHID, _NHID), lambda i: (0, 0)),
            pl.BlockSpec((_NHID, 16), lambda i: (0, 0)),
            pl.BlockSpec((_NHID, 16), lambda i: (0, 0)),
        ],
        out_specs=[
            pl.BlockSpec((blk, _NHID), lambda i: (i, 0)),
            pl.BlockSpec((blk, 16), lambda i: (i, 0)),
            pl.BlockSpec((blk, 16), lambda i: (i, 0)),
        ],
        out_shape=[
            jax.ShapeDtypeStruct((_N, _NHID), jnp.float32),
            jax.ShapeDtypeStruct((_N, 16), jnp.float32),
            jax.ShapeDtypeStruct((_N, 16), jnp.float32),
        ],
    )(x, wcat, a1p, a2p)


def _edge_body(wh_hbm, s1_hbm, s2_hbm, src_hbm, dst_hbm,
               msg_out, den_out,
               whv0, whv1, whv2,
               s1v0, s1v1, s2v0, s2v1, gv0, gv1,
               sidx0, sidx1, didx0, didx1, didxs0, didxs1,
               msg_acc, den_acc,
               semi0, semi1, semg0, semg1, semg2, sems0, sems1):
    c = lax.axis_index("c")
    s = lax.axis_index("s")

    WH = (whv0, whv1, whv2)
    SEMG = (semg0, semg1, semg2)
    S1 = (s1v0, s1v1)
    S2 = (s2v0, s2v1)
    GV = (gv0, gv1)
    SI = (sidx0, sidx1)
    DI = (didx0, didx1)
    DS = (didxs0, didxs1)
    SEMI = (semi0, semi1)
    SEMS = (sems0, sems1)

    z16 = jnp.zeros((16,), jnp.float32)

    @plsc.parallel_loop(0, _CH, step=1, unroll=4)
    def _zrow(e):
        for k in range(_NHID // 16):
            whv0[e, pl.ds(16 * k, 16)] = z16
        gv0[e, :] = z16

    r0 = s * _RPT

    @pl.when(s < _NS - 1)
    def _zero_full():
        for k in range(_RPT // _CH):
            pltpu.sync_copy(whv0, msg_acc.at[pl.ds(r0 + k * _CH, _CH)])
            pltpu.sync_copy(gv0, den_acc.at[pl.ds(r0 + k * _CH, _CH)])

    @pl.when(s == _NS - 1)
    def _zero_last():
        for k in range(_RPT_LAST // _CH):
            pltpu.sync_copy(whv0, msg_acc.at[pl.ds(r0 + k * _CH, _CH)])
            pltpu.sync_copy(gv0, den_acc.at[pl.ds(r0 + k * _CH, _CH)])

    plsc.subcore_barrier()

    lanes = lax.iota(jnp.int32, 16)
    headmask = lanes < _NHEADS
    ebase = (c * _NS + s) * _EPW

    def idx_start(j, r2):
        off = jnp.minimum(ebase + j * _CH, _E - _CH)
        pltpu.async_copy(src_hbm.at[pl.ds(off, _CH)], SI[r2], SEMI[r2])
        pltpu.async_copy(dst_hbm.at[pl.ds(off, _CH)], DI[r2], SEMI[r2])

    def idx_wait(r2):
        pltpu.make_async_copy(src_hbm.at[pl.ds(0, _CH)], SI[r2], SEMI[r2]).wait()
        pltpu.make_async_copy(dst_hbm.at[pl.ds(0, _CH)], DI[r2], SEMI[r2]).wait()

    def gat_start(r2, r3):
        pltpu.async_copy(s1_hbm.at[SI[r2]], S1[r2], SEMG[r3])
        pltpu.async_copy(s2_hbm.at[DI[r2]], S2[r2], SEMG[r3])
        pltpu.async_copy(wh_hbm.at[SI[r2]], WH[r3], SEMG[r3])

    def gat_wait(r2, r3):
        pltpu.make_async_copy(s1_hbm.at[SI[r2]], S1[r2], SEMG[r3]).wait()
        pltpu.make_async_copy(s2_hbm.at[DI[r2]], S2[r2], SEMG[r3]).wait()
        pltpu.make_async_copy(wh_hbm.at[SI[r2]], WH[r3], SEMG[r3]).wait()

    def compute(r2, r3):
        s1v, s2v, whv, gv, didxs = S1[r2], S2[r2], WH[r3], GV[r2], DS[r2]

        for k in range(0, _CH, 16):
            didxs[pl.ds(k, 16)] = DI[r2][pl.ds(k, 16)]

        @plsc.parallel_loop(0, _CH, step=1, unroll=4)
        def _edges(e):
            t = s1v[e, :] + s2v[e, :]
            t = jnp.maximum(t, _ALPHA * t)
            g = jnp.where(headmask, jnp.exp(t), 0.0)
            gv[e, :] = g
            for h in range(_NHEADS):
                gh = lax.gather(
                    g, jnp.full((16, 1), h, jnp.int32),
                    lax.GatherDimensionNumbers(
                        offset_dims=(), collapsed_slice_dims=(0,),
                        start_index_map=(0,)),
                    slice_sizes=(1,),
                    mode=lax.GatherScatterMode.PROMISE_IN_BOUNDS)
                w = whv[e, pl.ds(_DHEAD * h, _DHEAD)]
                whv[e, pl.ds(_DHEAD * h, _DHEAD)] = w * gh

    def scat_start(r2, r3):
        pltpu.async_copy(WH[r3], msg_acc.at[DS[r2]], SEMS[r2], add=True)
        pltpu.async_copy(GV[r2], den_acc.at[DS[r2]], SEMS[r2], add=True)

    def scat_wait(r2, r3):
        pltpu.make_async_copy(WH[r3], msg_acc.at[DS[r2]], SEMS[r2]).wait()
        pltpu.make_async_copy(GV[r2], den_acc.at[DS[r2]], SEMS[r2]).wait()

    def slot(jdyn, jo, do_next_gat, do_next_idx, guard_scat):
        r3, r2 = jo % 3, jo % 2
        r3n, r2n = (jo + 1) % 3, (jo + 1) % 2
        gat_wait(r2, r3)
        if guard_scat:
            @pl.when(jdyn >= 2)
            def _():
                scat_wait(r2, r3n)
        else:
            scat_wait(r2, r3n)
        if do_next_gat:
            idx_wait(r2n)
            gat_start(r2n, r3n)
        compute(r2, r3)
        scat_start(r2, r3)
        if do_next_idx:
            idx_start(jdyn + 2, r2)

    idx_start(0, 0)
    idx_wait(0)
    gat_start(0, 0)
    idx_start(1, 1)

    def six_body(k, carry):
        j = k * 6
        for jo in range(6):
            slot(j + jo, jo, True, True, True)
        return carry

    lax.fori_loop(0, _STEADY, six_body, 0)
    for jo in range(5):
        slot(_STEADY * 6 + jo, jo, jo < 4, jo < 3, False)
    scat_wait(1, 0)
    scat_wait(0, 1)
    plsc.subcore_barrier()

    @pl.when(s < _NS - 1)
    def _out_full():
        pltpu.sync_copy(msg_acc.at[pl.ds(r0, _RPT)], msg_out.at[c, pl.ds(r0, _RPT)])
        pltpu.sync_copy(den_acc.at[pl.ds(r0, _RPT)], den_out.at[c, pl.ds(r0, _RPT)])

    @pl.when(s == _NS - 1)
    def _out_last():
        pltpu.sync_copy(msg_acc.at[pl.ds(r0, _RPT_LAST)],
                        msg_out.at[c, pl.ds(r0, _RPT_LAST)])
        pltpu.sync_copy(den_acc.at[pl.ds(r0, _RPT_LAST)],
                        den_out.at[c, pl.ds(r0, _RPT_LAST)])


def _edge(wh, s1p, s2p, src, dst):
    mesh = plsc.VectorSubcoreMesh(core_axis_name="c", subcore_axis_name="s")
    run = functools.partial(
        pl.kernel,
        mesh=mesh,
        out_type=(
            jax.ShapeDtypeStruct((_NC, _N, _NHID), jnp.float32),
            jax.ShapeDtypeStruct((_NC, _N, 16), jnp.float32),
        ),
        scratch_types=[
            pltpu.VMEM((_CH, _NHID), jnp.float32),
            pltpu.VMEM((_CH, _NHID), jnp.float32),
            pltpu.VMEM((_CH, _NHID), jnp.float32),
            pltpu.VMEM((_CH, 16), jnp.float32),
            pltpu.VMEM((_CH, 16), jnp.float32),
            pltpu.VMEM((_CH, 16), jnp.float32),
            pltpu.VMEM((_CH, 16), jnp.float32),
            pltpu.VMEM((_CH, 16), jnp.float32),
            pltpu.VMEM((_CH, 16), jnp.float32),
            pltpu.VMEM((_CH,), jnp.int32),
            pltpu.VMEM((_CH,), jnp.int32),
            pltpu.VMEM((_CH,), jnp.int32),
            pltpu.VMEM((_CH,), jnp.int32),
            pltpu.VMEM((_CH,), jnp.int32),
            pltpu.VMEM((_CH,), jnp.int32),
            pltpu.VMEM_SHARED((_N, _NHID), jnp.float32),
            pltpu.VMEM_SHARED((_N, 16), jnp.float32),
            pltpu.SemaphoreType.DMA,
            pltpu.SemaphoreType.DMA,
            pltpu.SemaphoreType.DMA,
            pltpu.SemaphoreType.DMA,
            pltpu.SemaphoreType.DMA,
            pltpu.SemaphoreType.DMA,
            pltpu.SemaphoreType.DMA,
        ],
        compiler_params=pltpu.CompilerParams(use_tc_tiling_on_sc=False),
    )(_edge_body)
    return run(wh, s1p, s2p, src, dst)


def _gi_body(x_ref, wih_ref, bih_ref, gi_ref):
    gi_ref[...] = (jnp.dot(x_ref[...], wih_ref[...],
                           preferred_element_type=jnp.float32) + bih_ref[...])


def _gi(x, wih, bih):
    blk = 2000
    grid = (_N // blk,)
    return pl.pallas_call(
        _gi_body,
        grid=grid,
        in_specs=[
            pl.BlockSpec((blk, _NHID), lambda i: (i, 0)),
            pl.BlockSpec((_NHID, 3 * _NHID), lambda i: (0, 0)),
            pl.BlockSpec((1, 3 * _NHID), lambda i: (0, 0)),
        ],
        out_specs=pl.BlockSpec((blk, 3 * _NHID), lambda i: (i, 0)),
        out_shape=jax.ShapeDtypeStruct((_N, 3 * _NHID), jnp.float32),
    )(x, wih, bih)


def _gru_body(gi_ref, msg_ref, den_ref, whh_ref, bhh_ref, rmat_ref, out_ref):
    msg = msg_ref[0] + msg_ref[1]
    den = den_ref[0] + den_ref[1]
    den_rep = jnp.dot(den, rmat_ref[...], preferred_element_type=jnp.float32)
    hcat = jnp.where(den_rep > 0.0, msg / den_rep, 0.0)
    gi = gi_ref[...]
    gh = jnp.dot(hcat, whh_ref[...], preferred_element_type=jnp.float32) + bhh_ref[...]
    r = jax.nn.sigmoid(gi[:, 0:_NHID] + gh[:, 0:_NHID])
    z = jax.nn.sigmoid(gi[:, _NHID:2 * _NHID] + gh[:, _NHID:2 * _NHID])
    n = jnp.tanh(gi[:, 2 * _NHID:] + r * gh[:, 2 * _NHID:])
    out_ref[...] = (1.0 - z) * n + z * hcat


def _gru(gi, msg2, den2, whh, bhh, rmat):
    blk = 2000
    grid = (_N // blk,)
    return pl.pallas_call(
        _gru_body,
        grid=grid,
        in_specs=[
            pl.BlockSpec((blk, 3 * _NHID), lambda i: (i, 0)),
            pl.BlockSpec((_NC, blk, _NHID), lambda i: (0, i, 0)),
            pl.BlockSpec((_NC, blk, 16), lambda i: (0, i, 0)),
            pl.BlockSpec((_NHID, 3 * _NHID), lambda i: (0, 0)),
            pl.BlockSpec((1, 3 * _NHID), lambda i: (0, 0)),
            pl.BlockSpec((16, _NHID), lambda i: (0, 0)),
        ],
        out_specs=pl.BlockSpec((blk, _NHID), lambda i: (i, 0)),
        out_shape=jax.ShapeDtypeStruct((_N, _NHID), jnp.float32),
    )(gi, msg2, den2, whh, bhh, rmat)


def kernel(x, edge_index, W, a, W_ih, W_hh, b_ih, b_hh):
    wcat = jnp.transpose(W, (1, 0, 2)).reshape(_NHID, _NHID)
    a1 = a[:, :_DHEAD, 0]
    a2 = a[:, _DHEAD:, 0]
    eye = jnp.eye(_NHEADS, dtype=jnp.float32)
    a1p = jnp.pad((a1[:, :, None] * eye[:, None, :]).reshape(_NHID, _NHEADS),
                  ((0, 0), (0, 16 - _NHEADS)))
    a2p = jnp.pad((a2[:, :, None] * eye[:, None, :]).reshape(_NHID, _NHEADS),
                  ((0, 0), (0, 16 - _NHEADS)))
    rmat = (jnp.arange(_NHID)[None, :] // _DHEAD
            == jnp.arange(16)[:, None]).astype(jnp.float32)

    src = edge_index[0]
    dst = edge_index[1]

    wh, s1p, s2p = _prep(x, wcat, a1p, a2p)
    bih = b_ih.reshape(1, 3 * _NHID)
    bhh = b_hh.reshape(1, 3 * _NHID)
    msg2, den2 = _edge(wh, s1p, s2p, src, dst)
    gi = _gi(x, W_ih, bih)
    return _gru(gi, msg2, den2, W_hh, bhh, rmat)


# speedup vs baseline: 1.2775x; 1.0788x over previous
import functools

import jax
import jax.numpy as jnp
from jax import lax
from jax.experimental import pallas as pl
from jax.experimental.pallas import tpu as pltpu
from jax.experimental.pallas import tpu_sc as plsc

_N = 10000
_E = 320000
_NHID = 128
_NHEADS = 8
_DHEAD = _NHID // _NHEADS
_ALPHA = 0.2

_NC = 2
_NS = 16
_NW = _NC * _NS
_EPW = _E // _NW
_CH = 80
_NCH = _EPW // _CH
_STEADY = (_NCH - 5) // 6
_RPT = 640
_RPT_LAST = _N - (_NS - 1) * _RPT


def _prep_body(x_ref, wcat_ref, a1_ref, a2_ref, wh_ref, s1_ref, s2_ref):
    xb = x_ref[...]
    wh = jnp.dot(xb, wcat_ref[...], preferred_element_type=jnp.float32)
    wh_ref[...] = wh
    s1_ref[...] = jnp.dot(wh, a1_ref[...], preferred_element_type=jnp.float32)
    s2_ref[...] = jnp.dot(wh, a2_ref[...], preferred_element_type=jnp.float32)


def _prep(x, wcat, a1p, a2p):
    blk = 1000
    grid = (_N // blk,)
    return pl.pallas_call(
        _prep_body,
        grid=grid,
        in_specs=[
            pl.BlockSpec((blk, _NHID), lambda i: (i, 0)),
            pl.BlockSpec((_NHID, _NHID), lambda i: (0, 0)),
            pl.BlockSpec((_NHID, 16), lambda i: (0, 0)),
            pl.BlockSpec((_NHID, 16), lambda i: (0, 0)),
        ],
        out_specs=[
            pl.BlockSpec((blk, _NHID), lambda i: (i, 0)),
            pl.BlockSpec((blk, 16), lambda i: (i, 0)),
            pl.BlockSpec((blk, 16), lambda i: (i, 0)),
        ],
        out_shape=[
            jax.ShapeDtypeStruct((_N, _NHID), jnp.float32),
            jax.ShapeDtypeStruct((_N, 16), jnp.float32),
            jax.ShapeDtypeStruct((_N, 16), jnp.float32),
        ],
    )(x, wcat, a1p, a2p)


def _edge_body(wh_hbm, s1_hbm, s2_hbm, src_hbm, dst_hbm,
               msg_out, den_out,
               whv0, whv1, whv2,
               s1v0, s1v1, s2v0, s2v1, gv0, gv1,
               sidx0, sidx1, didx0, didx1, didxs0, didxs1,
               msg_acc, den_acc,
               semi0, semi1, semg0, semg1, semg2, sems0, sems1):
    c = lax.axis_index("c")
    s = lax.axis_index("s")

    WH = (whv0, whv1, whv2)
    SEMG = (semg0, semg1, semg2)
    S1 = (s1v0, s1v1)
    S2 = (s2v0, s2v1)
    GV = (gv0, gv1)
    SI = (sidx0, sidx1)
    DI = (didx0, didx1)
    DS = (didxs0, didxs1)
    SEMI = (semi0, semi1)
    SEMS = (sems0, sems1)

    z16 = jnp.zeros((16,), jnp.float32)

    @plsc.parallel_loop(0, _CH, step=1, unroll=4)
    def _zrow(e):
        for k in range(_NHID // 16):
            whv0[e, pl.ds(16 * k, 16)] = z16
        gv0[e, :] = z16

    r0 = s * _RPT

    @pl.when(s < _NS - 1)
    def _zero_full():
        for k in range(_RPT // _CH):
            pltpu.sync_copy(whv0, msg_acc.at[pl.ds(r0 + k * _CH, _CH)])
            pltpu.sync_copy(gv0, den_acc.at[pl.ds(r0 + k * _CH, _CH)])

    @pl.when(s == _NS - 1)
    def _zero_last():
        for k in range(_RPT_LAST // _CH):
            pltpu.sync_copy(whv0, msg_acc.at[pl.ds(r0 + k * _CH, _CH)])
            pltpu.sync_copy(gv0, den_acc.at[pl.ds(r0 + k * _CH, _CH)])

    plsc.subcore_barrier()

    ebase = (c * _NS + s) * _EPW

    def idx_start(j, r2):
        off = jnp.minimum(ebase + j * _CH, _E - _CH)
        pltpu.async_copy(src_hbm.at[pl.ds(off, _CH)], SI[r2], SEMI[r2])
        pltpu.async_copy(dst_hbm.at[pl.ds(off, _CH)], DI[r2], SEMI[r2])

    def idx_wait(r2):
        pltpu.make_async_copy(src_hbm.at[pl.ds(0, _CH)], SI[r2], SEMI[r2]).wait()
        pltpu.make_async_copy(dst_hbm.at[pl.ds(0, _CH)], DI[r2], SEMI[r2]).wait()

    def gat_start(r2, r3):
        pltpu.async_copy(s1_hbm.at[SI[r2]], S1[r2], SEMG[r3])
        pltpu.async_copy(s2_hbm.at[DI[r2]], S2[r2], SEMG[r3])
        pltpu.async_copy(wh_hbm.at[SI[r2]], WH[r3], SEMG[r3])

    def gat_wait(r2, r3):
        pltpu.make_async_copy(s1_hbm.at[SI[r2]], S1[r2], SEMG[r3]).wait()
        pltpu.make_async_copy(s2_hbm.at[DI[r2]], S2[r2], SEMG[r3]).wait()
        pltpu.make_async_copy(wh_hbm.at[SI[r2]], WH[r3], SEMG[r3]).wait()

    def compute(r2, r3):
        s1v, s2v, whv, gv, didxs = S1[r2], S2[r2], WH[r3], GV[r2], DS[r2]

        for k in range(0, _CH, 16):
            didxs[pl.ds(k, 16)] = DI[r2][pl.ds(k, 16)]

        @plsc.parallel_loop(0, _CH, step=1, unroll=4)
        def _edges(e):
            t = s1v[e, :] + s2v[e, :]
            t = jnp.maximum(t, _ALPHA * t)
            g = jnp.exp(t)
            gv[e, :] = g
            for h in range(_NHEADS):
                gh = lax.gather(
                    g, jnp.full((16, 1), h, jnp.int32),
                    lax.GatherDimensionNumbers(
                        offset_dims=(), collapsed_slice_dims=(0,),
                        start_index_map=(0,)),
                    slice_sizes=(1,),
                    mode=lax.GatherScatterMode.PROMISE_IN_BOUNDS)
                w = whv[e, pl.ds(_DHEAD * h, _DHEAD)]
                whv[e, pl.ds(_DHEAD * h, _DHEAD)] = w * gh

    def scat_start(r2, r3):
        pltpu.async_copy(WH[r3], msg_acc.at[DS[r2]], SEMS[r2], add=True)
        pltpu.async_copy(GV[r2], den_acc.at[DS[r2]], SEMS[r2], add=True)

    def scat_wait(r2, r3):
        pltpu.make_async_copy(WH[r3], msg_acc.at[DS[r2]], SEMS[r2]).wait()
        pltpu.make_async_copy(GV[r2], den_acc.at[DS[r2]], SEMS[r2]).wait()

    def slot(jdyn, jo, do_next_gat, do_next_idx, guard_scat):
        r3, r2 = jo % 3, jo % 2
        r3n, r2n = (jo + 1) % 3, (jo + 1) % 2
        if guard_scat:
            @pl.when(jdyn >= 2)
            def _():
                scat_wait(r2, r3n)
        else:
            scat_wait(r2, r3n)
        if do_next_gat:
            idx_wait(r2n)
            gat_start(r2n, r3n)
        gat_wait(r2, r3)
        compute(r2, r3)
        scat_start(r2, r3)
        if do_next_idx:
            idx_start(jdyn + 2, r2)

    idx_start(0, 0)
    idx_wait(0)
    gat_start(0, 0)
    idx_start(1, 1)

    def six_body(k, carry):
        j = k * 6
        for jo in range(6):
            slot(j + jo, jo, True, True, True)
        return carry

    lax.fori_loop(0, _STEADY, six_body, 0)
    for jo in range(5):
        slot(_STEADY * 6 + jo, jo, jo < 4, jo < 3, False)
    scat_wait(1, 0)
    scat_wait(0, 1)
    plsc.subcore_barrier()

    @pl.when(s < _NS - 1)
    def _out_full():
        pltpu.sync_copy(msg_acc.at[pl.ds(r0, _RPT)], msg_out.at[c, pl.ds(r0, _RPT)])
        pltpu.sync_copy(den_acc.at[pl.ds(r0, _RPT)], den_out.at[c, pl.ds(r0, _RPT)])

    @pl.when(s == _NS - 1)
    def _out_last():
        pltpu.sync_copy(msg_acc.at[pl.ds(r0, _RPT_LAST)],
                        msg_out.at[c, pl.ds(r0, _RPT_LAST)])
        pltpu.sync_copy(den_acc.at[pl.ds(r0, _RPT_LAST)],
                        den_out.at[c, pl.ds(r0, _RPT_LAST)])


def _edge(wh, s1p, s2p, src, dst):
    mesh = plsc.VectorSubcoreMesh(core_axis_name="c", subcore_axis_name="s")
    run = functools.partial(
        pl.kernel,
        mesh=mesh,
        out_type=(
            jax.ShapeDtypeStruct((_NC, _N, _NHID), jnp.float32),
            jax.ShapeDtypeStruct((_NC, _N, 16), jnp.float32),
        ),
        scratch_types=[
            pltpu.VMEM((_CH, _NHID), jnp.float32),
            pltpu.VMEM((_CH, _NHID), jnp.float32),
            pltpu.VMEM((_CH, _NHID), jnp.float32),
            pltpu.VMEM((_CH, 16), jnp.float32),
            pltpu.VMEM((_CH, 16), jnp.float32),
            pltpu.VMEM((_CH, 16), jnp.float32),
            pltpu.VMEM((_CH, 16), jnp.float32),
            pltpu.VMEM((_CH, 16), jnp.float32),
            pltpu.VMEM((_CH, 16), jnp.float32),
            pltpu.VMEM((_CH,), jnp.int32),
            pltpu.VMEM((_CH,), jnp.int32),
            pltpu.VMEM((_CH,), jnp.int32),
            pltpu.VMEM((_CH,), jnp.int32),
            pltpu.VMEM((_CH,), jnp.int32),
            pltpu.VMEM((_CH,), jnp.int32),
            pltpu.VMEM_SHARED((_N, _NHID), jnp.float32),
            pltpu.VMEM_SHARED((_N, 16), jnp.float32),
            pltpu.SemaphoreType.DMA,
            pltpu.SemaphoreType.DMA,
            pltpu.SemaphoreType.DMA,
            pltpu.SemaphoreType.DMA,
            pltpu.SemaphoreType.DMA,
            pltpu.SemaphoreType.DMA,
            pltpu.SemaphoreType.DMA,
        ],
        compiler_params=pltpu.CompilerParams(use_tc_tiling_on_sc=False),
    )(_edge_body)
    return run(wh, s1p, s2p, src, dst)


def _gi_body(x_ref, wih_ref, bih_ref, gi_ref):
    gi_ref[...] = (jnp.dot(x_ref[...], wih_ref[...],
                           preferred_element_type=jnp.float32) + bih_ref[...])


def _gi(x, wih, bih):
    blk = 2000
    grid = (_N // blk,)
    return pl.pallas_call(
        _gi_body,
        grid=grid,
        in_specs=[
            pl.BlockSpec((blk, _NHID), lambda i: (i, 0)),
            pl.BlockSpec((_NHID, 3 * _NHID), lambda i: (0, 0)),
            pl.BlockSpec((1, 3 * _NHID), lambda i: (0, 0)),
        ],
        out_specs=pl.BlockSpec((blk, 3 * _NHID), lambda i: (i, 0)),
        out_shape=jax.ShapeDtypeStruct((_N, 3 * _NHID), jnp.float32),
    )(x, wih, bih)


def _gru_body(gi_ref, msg_ref, den_ref, whh_ref, bhh_ref, rmat_ref, out_ref):
    msg = msg_ref[0] + msg_ref[1]
    den = den_ref[0] + den_ref[1]
    den_rep = jnp.dot(den, rmat_ref[...], preferred_element_type=jnp.float32)
    hcat = jnp.where(den_rep > 0.0, msg / den_rep, 0.0)
    gi = gi_ref[...]
    gh = jnp.dot(hcat, whh_ref[...], preferred_element_type=jnp.float32) + bhh_ref[...]
    r = jax.nn.sigmoid(gi[:, 0:_NHID] + gh[:, 0:_NHID])
    z = jax.nn.sigmoid(gi[:, _NHID:2 * _NHID] + gh[:, _NHID:2 * _NHID])
    n = jnp.tanh(gi[:, 2 * _NHID:] + r * gh[:, 2 * _NHID:])
    out_ref[...] = (1.0 - z) * n + z * hcat


def _gru(gi, msg2, den2, whh, bhh, rmat):
    blk = 2000
    grid = (_N // blk,)
    return pl.pallas_call(
        _gru_body,
        grid=grid,
        in_specs=[
            pl.BlockSpec((blk, 3 * _NHID), lambda i: (i, 0)),
            pl.BlockSpec((_NC, blk, _NHID), lambda i: (0, i, 0)),
            pl.BlockSpec((_NC, blk, 16), lambda i: (0, i, 0)),
            pl.BlockSpec((_NHID, 3 * _NHID), lambda i: (0, 0)),
            pl.BlockSpec((1, 3 * _NHID), lambda i: (0, 0)),
            pl.BlockSpec((16, _NHID), lambda i: (0, 0)),
        ],
        out_specs=pl.BlockSpec((blk, _NHID), lambda i: (i, 0)),
        out_shape=jax.ShapeDtypeStruct((_N, _NHID), jnp.float32),
    )(gi, msg2, den2, whh, bhh, rmat)


def kernel(x, edge_index, W, a, W_ih, W_hh, b_ih, b_hh):
    wcat = jnp.transpose(W, (1, 0, 2)).reshape(_NHID, _NHID)
    a1 = a[:, :_DHEAD, 0]
    a2 = a[:, _DHEAD:, 0]
    eye = jnp.eye(_NHEADS, dtype=jnp.float32)
    a1p = jnp.pad((a1[:, :, None] * eye[:, None, :]).reshape(_NHID, _NHEADS),
                  ((0, 0), (0, 16 - _NHEADS)))
    a2p = jnp.pad((a2[:, :, None] * eye[:, None, :]).reshape(_NHID, _NHEADS),
                  ((0, 0), (0, 16 - _NHEADS)))
    rmat = (jnp.arange(_NHID)[None, :] // _DHEAD
            == jnp.arange(16)[:, None]).astype(jnp.float32)

    src = edge_index[0]
    dst = edge_index[1]

    wh, s1p, s2p = _prep(x, wcat, a1p, a2p)
    bih = b_ih.reshape(1, 3 * _NHID)
    bhh = b_hh.reshape(1, 3 * _NHID)
    msg2, den2 = _edge(wh, s1p, s2p, src, dst)
    gi = _gi(x, W_ih, bih)
    return _gru(gi, msg2, den2, W_hh, bhh, rmat)


# speedup vs baseline: 1.2990x; 1.0169x over previous
import functools

import jax
import jax.numpy as jnp
from jax import lax
from jax.experimental import pallas as pl
from jax.experimental.pallas import tpu as pltpu
from jax.experimental.pallas import tpu_sc as plsc

_N = 10000
_E = 320000
_NHID = 128
_NHEADS = 8
_DHEAD = _NHID // _NHEADS
_ALPHA = 0.2

_NC = 2
_NS = 16
_NW = _NC * _NS
_EPW = _E // _NW
_CH = 80
_NCH = _EPW // _CH
_STEADY = (_NCH - 5) // 6
_RPT = 640
_RPT_LAST = _N - (_NS - 1) * _RPT


def _prep_body(x_ref, wcat_ref, a1_ref, a2_ref, wh_ref, s1_ref, s2_ref):
    xb = x_ref[...]
    wh = jnp.dot(xb, wcat_ref[...], preferred_element_type=jnp.float32)
    wh_ref[...] = wh
    s1_ref[...] = jnp.dot(wh, a1_ref[...], preferred_element_type=jnp.float32)
    s2_ref[...] = jnp.dot(wh, a2_ref[...], preferred_element_type=jnp.float32)


def _prep(x, wcat, a1p, a2p):
    blk = 2000
    grid = (_N // blk,)
    return pl.pallas_call(
        _prep_body,
        grid=grid,
        in_specs=[
            pl.BlockSpec((blk, _NHID), lambda i: (i, 0)),
            pl.BlockSpec((_NHID, _NHID), lambda i: (0, 0)),
            pl.BlockSpec((_NHID, 16), lambda i: (0, 0)),
            pl.BlockSpec((_NHID, 16), lambda i: (0, 0)),
        ],
        out_specs=[
            pl.BlockSpec((blk, _NHID), lambda i: (i, 0)),
            pl.BlockSpec((blk, 16), lambda i: (i, 0)),
            pl.BlockSpec((blk, 16), lambda i: (i, 0)),
        ],
        out_shape=[
            jax.ShapeDtypeStruct((_N, _NHID), jnp.float32),
            jax.ShapeDtypeStruct((_N, 16), jnp.float32),
            jax.ShapeDtypeStruct((_N, 16), jnp.float32),
        ],
    )(x, wcat, a1p, a2p)


def _edge_body(wh_hbm, s1_hbm, s2_hbm, src_hbm, dst_hbm,
               msg_out, den_out,
               whv0, whv1, whv2,
               s1v0, s1v1, s2v0, s2v1, gv0, gv1,
               sidx0, sidx1, didx0, didx1, didxs0, didxs1,
               msg_acc, den_acc,
               semi0, semi1, semg0, semg1, semg2, sems0, sems1):
    c = lax.axis_index("c")
    s = lax.axis_index("s")

    WH = (whv0, whv1, whv2)
    SEMG = (semg0, semg1, semg2)
    S1 = (s1v0, s1v1)
    S2 = (s2v0, s2v1)
    GV = (gv0, gv1)
    SI = (sidx0, sidx1)
    DI = (didx0, didx1)
    DS = (didxs0, didxs1)
    SEMI = (semi0, semi1)
    SEMS = (sems0, sems1)

    z16 = jnp.zeros((16,), jnp.float32)

    @plsc.parallel_loop(0, _CH, step=1, unroll=4)
    def _zrow(e):
        for k in range(_NHID // 16):
            whv0[e, pl.ds(16 * k, 16)] = z16
        gv0[e, :] = z16

    r0 = s * _RPT

    @pl.when(s < _NS - 1)
    def _zero_full():
        for k in range(_RPT // _CH):
            pltpu.sync_copy(whv0, msg_acc.at[pl.ds(r0 + k * _CH, _CH)])
            pltpu.sync_copy(gv0, den_acc.at[pl.ds(r0 + k * _CH, _CH)])

    @pl.when(s == _NS - 1)
    def _zero_last():
        for k in range(_RPT_LAST // _CH):
            pltpu.sync_copy(whv0, msg_acc.at[pl.ds(r0 + k * _CH, _CH)])
            pltpu.sync_copy(gv0, den_acc.at[pl.ds(r0 + k * _CH, _CH)])

    plsc.subcore_barrier()

    ebase = (c * _NS + s) * _EPW

    def idx_start(j, r2):
        off = jnp.minimum(ebase + j * _CH, _E - _CH)
        pltpu.async_copy(src_hbm.at[pl.ds(off, _CH)], SI[r2], SEMI[r2])
        pltpu.async_copy(dst_hbm.at[pl.ds(off, _CH)], DI[r2], SEMI[r2])

    def idx_wait(r2):
        pltpu.make_async_copy(src_hbm.at[pl.ds(0, _CH)], SI[r2], SEMI[r2]).wait()
        pltpu.make_async_copy(dst_hbm.at[pl.ds(0, _CH)], DI[r2], SEMI[r2]).wait()

    def gat_start(r2, r3):
        pltpu.async_copy(s1_hbm.at[SI[r2]], S1[r2], SEMG[r3])
        pltpu.async_copy(s2_hbm.at[DI[r2]], S2[r2], SEMG[r3])
        pltpu.async_copy(wh_hbm.at[SI[r2]], WH[r3], SEMG[r3])

    def gat_wait(r2, r3):
        pltpu.make_async_copy(s1_hbm.at[SI[r2]], S1[r2], SEMG[r3]).wait()
        pltpu.make_async_copy(s2_hbm.at[DI[r2]], S2[r2], SEMG[r3]).wait()
        pltpu.make_async_copy(wh_hbm.at[SI[r2]], WH[r3], SEMG[r3]).wait()

    def compute(r2, r3):
        s1v, s2v, whv, gv, didxs = S1[r2], S2[r2], WH[r3], GV[r2], DS[r2]

        for k in range(0, _CH, 16):
            didxs[pl.ds(k, 16)] = DI[r2][pl.ds(k, 16)]

        @plsc.parallel_loop(0, _CH, step=1, unroll=4)
        def _edges(e):
            t = s1v[e, :] + s2v[e, :]
            t = jnp.maximum(t, _ALPHA * t)
            g = jnp.exp(t)
            gv[e, :] = g
            for h in range(_NHEADS):
                gh = lax.gather(
                    g, jnp.full((16, 1), h, jnp.int32),
                    lax.GatherDimensionNumbers(
                        offset_dims=(), collapsed_slice_dims=(0,),
                        start_index_map=(0,)),
                    slice_sizes=(1,),
                    mode=lax.GatherScatterMode.PROMISE_IN_BOUNDS)
                w = whv[e, pl.ds(_DHEAD * h, _DHEAD)]
                whv[e, pl.ds(_DHEAD * h, _DHEAD)] = w * gh

    def scat_start(r2, r3):
        pltpu.async_copy(WH[r3], msg_acc.at[DS[r2]], SEMS[r2], add=True)
        pltpu.async_copy(GV[r2], den_acc.at[DS[r2]], SEMS[r2], add=True)

    def scat_wait(r2, r3):
        pltpu.make_async_copy(WH[r3], msg_acc.at[DS[r2]], SEMS[r2]).wait()
        pltpu.make_async_copy(GV[r2], den_acc.at[DS[r2]], SEMS[r2]).wait()

    def slot(jdyn, jo, do_next_gat, do_next_idx, guard_scat):
        r3, r2 = jo % 3, jo % 2
        r3n, r2n = (jo + 1) % 3, (jo + 1) % 2
        if guard_scat:
            @pl.when(jdyn >= 2)
            def _():
                scat_wait(r2, r3n)
        else:
            scat_wait(r2, r3n)
        if do_next_gat:
            idx_wait(r2n)
            gat_start(r2n, r3n)
        gat_wait(r2, r3)
        compute(r2, r3)
        scat_start(r2, r3)
        if do_next_idx:
            idx_start(jdyn + 2, r2)

    idx_start(0, 0)
    idx_wait(0)
    gat_start(0, 0)
    idx_start(1, 1)

    def six_body(k, carry):
        j = k * 6
        for jo in range(6):
            slot(j + jo, jo, True, True, True)
        return carry

    lax.fori_loop(0, _STEADY, six_body, 0)
    for jo in range(5):
        slot(_STEADY * 6 + jo, jo, jo < 4, jo < 3, False)
    scat_wait(1, 0)
    scat_wait(0, 1)
    plsc.subcore_barrier()

    @pl.when(s < _NS - 1)
    def _out_full():
        pltpu.sync_copy(msg_acc.at[pl.ds(r0, _RPT)], msg_out.at[c, pl.ds(r0, _RPT)])
        pltpu.sync_copy(den_acc.at[pl.ds(r0, _RPT)], den_out.at[c, pl.ds(r0, _RPT)])

    @pl.when(s == _NS - 1)
    def _out_last():
        pltpu.sync_copy(msg_acc.at[pl.ds(r0, _RPT_LAST)],
                        msg_out.at[c, pl.ds(r0, _RPT_LAST)])
        pltpu.sync_copy(den_acc.at[pl.ds(r0, _RPT_LAST)],
                        den_out.at[c, pl.ds(r0, _RPT_LAST)])


def _edge(wh, s1p, s2p, src, dst):
    mesh = plsc.VectorSubcoreMesh(core_axis_name="c", subcore_axis_name="s")
    run = functools.partial(
        pl.kernel,
        mesh=mesh,
        out_type=(
            jax.ShapeDtypeStruct((_NC, _N, _NHID), jnp.float32),
            jax.ShapeDtypeStruct((_NC, _N, 16), jnp.float32),
        ),
        scratch_types=[
            pltpu.VMEM((_CH, _NHID), jnp.float32),
            pltpu.VMEM((_CH, _NHID), jnp.float32),
            pltpu.VMEM((_CH, _NHID), jnp.float32),
            pltpu.VMEM((_CH, 16), jnp.float32),
            pltpu.VMEM((_CH, 16), jnp.float32),
            pltpu.VMEM((_CH, 16), jnp.float32),
            pltpu.VMEM((_CH, 16), jnp.float32),
            pltpu.VMEM((_CH, 16), jnp.float32),
            pltpu.VMEM((_CH, 16), jnp.float32),
            pltpu.VMEM((_CH,), jnp.int32),
            pltpu.VMEM((_CH,), jnp.int32),
            pltpu.VMEM((_CH,), jnp.int32),
            pltpu.VMEM((_CH,), jnp.int32),
            pltpu.VMEM((_CH,), jnp.int32),
            pltpu.VMEM((_CH,), jnp.int32),
            pltpu.VMEM_SHARED((_N, _NHID), jnp.float32),
            pltpu.VMEM_SHARED((_N, 16), jnp.float32),
            pltpu.SemaphoreType.DMA,
            pltpu.SemaphoreType.DMA,
            pltpu.SemaphoreType.DMA,
            pltpu.SemaphoreType.DMA,
            pltpu.SemaphoreType.DMA,
            pltpu.SemaphoreType.DMA,
            pltpu.SemaphoreType.DMA,
        ],
        compiler_params=pltpu.CompilerParams(use_tc_tiling_on_sc=False),
    )(_edge_body)
    return run(wh, s1p, s2p, src, dst)


def _gi_body(x_ref, wih_ref, bih_ref, gi_ref):
    gi_ref[...] = (jnp.dot(x_ref[...], wih_ref[...],
                           preferred_element_type=jnp.float32) + bih_ref[...])


def _gi(x, wih, bih):
    blk = 2000
    grid = (_N // blk,)
    return pl.pallas_call(
        _gi_body,
        grid=grid,
        in_specs=[
            pl.BlockSpec((blk, _NHID), lambda i: (i, 0)),
            pl.BlockSpec((_NHID, 3 * _NHID), lambda i: (0, 0)),
            pl.BlockSpec((1, 3 * _NHID), lambda i: (0, 0)),
        ],
        out_specs=pl.BlockSpec((blk, 3 * _NHID), lambda i: (i, 0)),
        out_shape=jax.ShapeDtypeStruct((_N, 3 * _NHID), jnp.float32),
    )(x, wih, bih)


def _gru_body(gi_ref, msg_ref, den_ref, whh_ref, bhh_ref, rmat_ref, out_ref):
    msg = msg_ref[0] + msg_ref[1]
    den = den_ref[0] + den_ref[1]
    den_rep = jnp.dot(den, rmat_ref[...], preferred_element_type=jnp.float32)
    hcat = jnp.where(den_rep > 0.0, msg / den_rep, 0.0)
    gi = gi_ref[...]
    gh = jnp.dot(hcat, whh_ref[...], preferred_element_type=jnp.float32) + bhh_ref[...]
    r = jax.nn.sigmoid(gi[:, 0:_NHID] + gh[:, 0:_NHID])
    z = jax.nn.sigmoid(gi[:, _NHID:2 * _NHID] + gh[:, _NHID:2 * _NHID])
    n = jnp.tanh(gi[:, 2 * _NHID:] + r * gh[:, 2 * _NHID:])
    out_ref[...] = (1.0 - z) * n + z * hcat


def _gru(gi, msg2, den2, whh, bhh, rmat):
    blk = 2000
    grid = (_N // blk,)
    return pl.pallas_call(
        _gru_body,
        grid=grid,
        in_specs=[
            pl.BlockSpec((blk, 3 * _NHID), lambda i: (i, 0)),
            pl.BlockSpec((_NC, blk, _NHID), lambda i: (0, i, 0)),
            pl.BlockSpec((_NC, blk, 16), lambda i: (0, i, 0)),
            pl.BlockSpec((_NHID, 3 * _NHID), lambda i: (0, 0)),
            pl.BlockSpec((1, 3 * _NHID), lambda i: (0, 0)),
            pl.BlockSpec((16, _NHID), lambda i: (0, 0)),
        ],
        out_specs=pl.BlockSpec((blk, _NHID), lambda i: (i, 0)),
        out_shape=jax.ShapeDtypeStruct((_N, _NHID), jnp.float32),
    )(gi, msg2, den2, whh, bhh, rmat)


def kernel(x, edge_index, W, a, W_ih, W_hh, b_ih, b_hh):
    wcat = jnp.transpose(W, (1, 0, 2)).reshape(_NHID, _NHID)
    a1 = a[:, :_DHEAD, 0]
    a2 = a[:, _DHEAD:, 0]
    eye = jnp.eye(_NHEADS, dtype=jnp.float32)
    a1p = jnp.pad((a1[:, :, None] * eye[:, None, :]).reshape(_NHID, _NHEADS),
                  ((0, 0), (0, 16 - _NHEADS)))
    a2p = jnp.pad((a2[:, :, None] * eye[:, None, :]).reshape(_NHID, _NHEADS),
                  ((0, 0), (0, 16 - _NHEADS)))
    rmat = (jnp.arange(_NHID)[None, :] // _DHEAD
            == jnp.arange(16)[:, None]).astype(jnp.float32)

    src = edge_index[0]
    dst = edge_index[1]

    wh, s1p, s2p = _prep(x, wcat, a1p, a2p)
    bih = b_ih.reshape(1, 3 * _NHID)
    bhh = b_hh.reshape(1, 3 * _NHID)
    msg2, den2 = _edge(wh, s1p, s2p, src, dst)
    gi = _gi(x, W_ih, bih)
    return _gru(gi, msg2, den2, W_hh, bhh, rmat)
